# baseline (speedup 1.0000x reference)
import functools

import jax
import jax.numpy as jnp
from jax import lax
from jax.experimental import pallas as pl
from jax.experimental.pallas import tpu as pltpu
from jax.experimental.pallas import tpu_sc as plsc

BATCH = 16384
D = 64
LANES = 16
NUM_WORKERS = 32
B_PER_W = BATCH // NUM_WORKERS
CHUNK = 128
NCHUNK = B_PER_W // CHUNK
GROUPS = CHUNK // LANES


def _sqrt_f32(x):
    i = lax.bitcast_convert_type(x, jnp.int32)
    y = lax.bitcast_convert_type(jnp.int32(0x5F3759DF) - (i >> 1), jnp.float32)
    for _ in range(3):
        y = y * (1.5 - 0.5 * x * y * y)
    return x * y


_GATHER_DNUMS = lax.GatherDimensionNumbers(
    offset_dims=(), collapsed_slice_dims=(0,), start_index_map=(0,))


def _rot(v, perm):
    return lax.gather(v, perm, _GATHER_DNUMS, (1,),
                      mode=lax.GatherScatterMode.PROMISE_IN_BOUNDS)


def _lane_total(v, perms):
    for p in perms:
        v = v + _rot(v, p)
    return v


_mesh = plsc.VectorSubcoreMesh(core_axis_name="c", subcore_axis_name="s")


@functools.partial(
    pl.kernel,
    mesh=_mesh,
    compiler_params=pltpu.CompilerParams(use_tc_tiling_on_sc=False),
    out_type=[
        jax.ShapeDtypeStruct((BATCH,), jnp.float32),
        jax.ShapeDtypeStruct((BATCH,), jnp.float32),
    ],
    scratch_types=[
        pltpu.VMEM((B_PER_W,), jnp.int32),
        pltpu.VMEM((B_PER_W,), jnp.int32),
        pltpu.VMEM((B_PER_W,), jnp.int32),
        pltpu.VMEM((B_PER_W,), jnp.int32),
        pltpu.VMEM((CHUNK, D), jnp.float32),
        pltpu.VMEM((CHUNK, D), jnp.float32),
        pltpu.VMEM((CHUNK, D), jnp.float32),
        pltpu.VMEM((CHUNK, D), jnp.float32),
        pltpu.VMEM((CHUNK, D), jnp.float32),
        pltpu.VMEM((CHUNK, D), jnp.float32),
        pltpu.VMEM((CHUNK, D), jnp.float32),
        pltpu.VMEM((CHUNK, D), jnp.float32),
        pltpu.VMEM((B_PER_W,), jnp.float32),
        pltpu.VMEM((B_PER_W,), jnp.float32),
        pltpu.SemaphoreType.DMA,
        pltpu.SemaphoreType.DMA,
    ],
)
def _transe_sc(heads_hbm, rels_hbm, tails_hbm, corrs_hbm, ent_hbm, rel_emb_hbm,
               pos_hbm, neg_hbm,
               h_idx, r_idx, t_idx, c_idx,
               hb0, hb1, rb0, rb1, tb0, tb1, cb0, cb1, pos_v, neg_v,
               sem0, sem1):
    wid = lax.axis_index("s") * 2 + lax.axis_index("c")
    base = wid * B_PER_W

    pltpu.sync_copy(heads_hbm.at[pl.ds(base, B_PER_W)], h_idx)
    pltpu.sync_copy(rels_hbm.at[pl.ds(base, B_PER_W)], r_idx)
    pltpu.sync_copy(tails_hbm.at[pl.ds(base, B_PER_W)], t_idx)
    pltpu.sync_copy(corrs_hbm.at[pl.ds(base, B_PER_W)], c_idx)

    sems = (sem0, sem1)
    bufs = ((hb0, rb0, tb0, cb0), (hb1, rb1, tb1, cb1))

    def fire(j):
        hbuf, rbuf, tbuf, cbuf = bufs[j % 2]
        sem = sems[j % 2]
        s = j * CHUNK
        return (
            pltpu.async_copy(ent_hbm.at[h_idx.at[pl.ds(s, CHUNK)]], hbuf, sem),
            pltpu.async_copy(rel_emb_hbm.at[r_idx.at[pl.ds(s, CHUNK)]], rbuf, sem),
            pltpu.async_copy(ent_hbm.at[t_idx.at[pl.ds(s, CHUNK)]], tbuf, sem),
            pltpu.async_copy(ent_hbm.at[c_idx.at[pl.ds(s, CHUNK)]], cbuf, sem),
        )

    lane = jax.lax.iota(jnp.int32, LANES)
    perms = tuple(((lane + r) % LANES)[:, None] for r in (8, 4, 2, 1))
    pending = fire(0)

    for j in range(NCHUNK):
        hbuf, rbuf, tbuf, cbuf = bufs[j % 2]
        if j + 1 < NCHUNK:
            nxt = fire(j + 1)
        for d in pending:
            d.wait()
        pending = nxt if j + 1 < NCHUNK else ()

        def group_body(g, _):
            row0 = g * LANES
            vec_p = jnp.zeros((LANES,), jnp.float32)
            vec_n = jnp.zeros((LANES,), jnp.float32)
            for i in range(LANES):
                r = row0 + i
                acc_p = jnp.zeros((LANES,), jnp.float32)
                acc_n = jnp.zeros((LANES,), jnp.float32)
                for k in range(D // LANES):
                    sl = pl.ds(k * LANES, LANES)
                    s = hbuf[r, sl] + rbuf[r, sl]
                    p = s - tbuf[r, sl]
                    n = s - cbuf[r, sl]
                    acc_p = acc_p + p * p
                    acc_n = acc_n + n * n
                vec_p = jnp.where(lane == i, _lane_total(acc_p, perms), vec_p)
                vec_n = jnp.where(lane == i, _lane_total(acc_n, perms), vec_n)
            off = pl.multiple_of(j * CHUNK, LANES) + row0
            pos_v[pl.ds(off, LANES)] = _sqrt_f32(vec_p)
            neg_v[pl.ds(off, LANES)] = _sqrt_f32(vec_n)
            return 0

        lax.fori_loop(0, GROUPS, group_body, 0)

    pltpu.sync_copy(pos_v, pos_hbm.at[pl.ds(base, B_PER_W)])
    pltpu.sync_copy(neg_v, neg_hbm.at[pl.ds(base, B_PER_W)])


@jax.jit
def kernel(heads, relations, tails, corrupted_tails, entity_emb, relation_emb):
    pos, neg = _transe_sc(
        heads.astype(jnp.int32),
        relations.astype(jnp.int32),
        tails.astype(jnp.int32),
        corrupted_tails.astype(jnp.int32),
        entity_emb,
        relation_emb,
    )
    return pos, neg
